# baseline (speedup 1.0000x reference)
import functools

import jax
import jax.numpy as jnp
from jax import lax
from jax.experimental import pallas as pl
from jax.experimental.pallas import tpu as pltpu
from jax.experimental.pallas import tpu_sc as plsc

BATCH = 16384
REPR = 32
L = 16
NC, NS = 2, 16
NW = NC * NS
BPW = BATCH // NW
GCHUNK = 128

_mesh = plsc.VectorSubcoreMesh(core_axis_name="c", subcore_axis_name="s")


@functools.partial(
    pl.kernel,
    mesh=_mesh,
    out_type=jax.ShapeDtypeStruct((BATCH,), jnp.float32),
    compiler_params=pltpu.CompilerParams(
        needs_layout_passes=False, use_tc_tiling_on_sc=False),
    scratch_types=[
        pltpu.VMEM((BPW,), jnp.int32),
        pltpu.VMEM((BPW, REPR), jnp.float32),
        pltpu.VMEM((BPW, REPR), jnp.float32),
        pltpu.VMEM((BPW,), jnp.float32),
        pltpu.SemaphoreType.DMA,
    ],
)
def _dist_kernel(rep_hbm, idx_hbm, table_hbm, out_hbm,
                 idx_v, emb_v, rep_v, out_v, sem):
    wid = lax.axis_index("s") * NC + lax.axis_index("c")
    base = wid * BPW

    pltpu.sync_copy(idx_hbm.at[pl.ds(base, BPW)], idx_v)
    copies = []
    for k in range(BPW // GCHUNK):
        copies.append(pltpu.async_copy(
            table_hbm.at[idx_v.at[pl.ds(k * GCHUNK, GCHUNK)]],
            emb_v.at[pl.ds(k * GCHUNK, GCHUNK)],
            sem,
        ))
    pltpu.sync_copy(rep_hbm.at[pl.ds(base, BPW)], rep_v)
    for c in copies:
        c.wait()

    def group(g, carry):
        rows = g * L + lax.iota(jnp.int32, L)
        acc = jnp.full((L,), 1e-12, jnp.float32)
        for d in range(REPR):
            col = jnp.full((L,), d, jnp.int32)
            e = plsc.load_gather(emb_v, [rows, col])
            r = plsc.load_gather(rep_v, [rows, col])
            df = e - r
            acc = acc + df * df
        i = lax.bitcast_convert_type(acc, jnp.int32)
        i = 0x5F3759DF - lax.shift_right_logical(i, 1)
        y = lax.bitcast_convert_type(i, jnp.float32)
        half = acc * jnp.float32(0.5)
        for _ in range(3):
            y = y * (jnp.float32(1.5) - half * y * y)
        out_v[pl.ds(g * L, L)] = acc * y
        return carry

    lax.fori_loop(0, BPW // L, group, 0)
    pltpu.sync_copy(out_v, out_hbm.at[pl.ds(base, BPW)])


def kernel(rep, oracle_structure, embedding_weight):
    idx = oracle_structure.astype(jnp.int32)
    return _dist_kernel(rep, idx, embedding_weight)


# speedup vs baseline: 3.5568x; 3.5568x over previous
import functools

import jax
import jax.numpy as jnp
from jax import lax
from jax.experimental import pallas as pl
from jax.experimental.pallas import tpu as pltpu
from jax.experimental.pallas import tpu_sc as plsc

BATCH = 16384
REPR = 32
L = 16
NC, NS = 2, 16
NW = NC * NS
BPW = BATCH // NW
CHUNK = 16
NCHUNK = BPW // CHUNK

_mesh = plsc.VectorSubcoreMesh(core_axis_name="c", subcore_axis_name="s")


@functools.partial(
    pl.kernel,
    mesh=_mesh,
    out_type=jax.ShapeDtypeStruct((BATCH,), jnp.float32),
    compiler_params=pltpu.CompilerParams(
        needs_layout_passes=False, use_tc_tiling_on_sc=True),
    scratch_types=[
        pltpu.VMEM((BPW,), jnp.int32),
        pltpu.VMEM((CHUNK * 4, 8, 128), jnp.float32),
        pltpu.VMEM((16, 8, 128), jnp.float32),
        pltpu.VMEM((BPW,), jnp.float32),
        pltpu.SemaphoreType.DMA,
    ],
)
def _dist_kernel(rep_t_hbm, idx_hbm, table3_hbm, out_hbm,
                 idx_v, emb_v, rep_v, out_v, sem):
    wid = lax.axis_index("s") * NC + lax.axis_index("c")
    base = wid * BPW

    pltpu.sync_copy(idx_hbm.at[pl.ds(base, BPW)], idx_v)
    for rb in range(4):
        for cb in range(4):
            pltpu.sync_copy(
                rep_t_hbm.at[pl.ds(rb * 8, 8), pl.ds(base + cb * 128, 128)],
                rep_v.at[rb * 4 + cb],
            )

    lane = lax.iota(jnp.int32, L)

    for c in range(NCHUNK):
        c0 = c * CHUNK
        vec = idx_v[pl.ds(c0, L)]
        descs = []
        for j in range(CHUNK):
            va = pl.multiple_of(
                lax.shift_left(lax.shift_right_logical(vec[j], 7), 7), 128)
            descs.append(pltpu.async_copy(
                table3_hbm.at[:, :, pl.ds(va, 128)],
                emb_v.at[pl.ds(j * 4, 4)],
                sem,
            ))
        for dsc in descs:
            dsc.wait()

        off = lax.bitwise_and(vec, jnp.int32(127))
        tile_base = lane * 4
        acc = jnp.full((L,), 1e-12, jnp.float32)
        for d in range(REPR):
            tile = tile_base + jnp.int32(d // 8)
            r = jnp.full((L,), d % 8, jnp.int32)
            e = plsc.load_gather(emb_v, [tile, r, off])
            rv = rep_v[(d // 8) * 4 + c // 8, d % 8, pl.ds((c % 8) * L, L)]
            df = e - rv
            acc = acc + df * df
        i = lax.bitcast_convert_type(acc, jnp.int32)
        i = 0x5F3759DF - lax.shift_right_logical(i, 1)
        y = lax.bitcast_convert_type(i, jnp.float32)
        half = acc * jnp.float32(0.5)
        for _ in range(3):
            y = y * (jnp.float32(1.5) - half * y * y)
        out_v[pl.ds(c0, L)] = acc * y

    pltpu.sync_copy(out_v, out_hbm.at[pl.ds(base, BPW)])


def kernel(rep, oracle_structure, embedding_weight):
    idx = oracle_structure.astype(jnp.int32)
    table3 = embedding_weight.T.reshape(4, 8, 1000000)
    return _dist_kernel(rep.T, idx, table3)


# speedup vs baseline: 4.2922x; 1.2068x over previous
import functools

import jax
import jax.numpy as jnp
from jax import lax
from jax.experimental import pallas as pl
from jax.experimental.pallas import tpu as pltpu
from jax.experimental.pallas import tpu_sc as plsc

BATCH = 16384
REPR = 32
L = 16
NC, NS = 2, 16
NW = NC * NS
BPW = BATCH // NW
H = 8
HPS = 16
NSUP = BPW // (H * HPS)

_mesh = plsc.VectorSubcoreMesh(core_axis_name="c", subcore_axis_name="s")


@functools.partial(
    pl.kernel,
    mesh=_mesh,
    out_type=jax.ShapeDtypeStruct((BATCH,), jnp.float32),
    compiler_params=pltpu.CompilerParams(
        needs_layout_passes=False, use_tc_tiling_on_sc=True),
    scratch_types=[
        pltpu.VMEM((BPW,), jnp.int32),
        pltpu.VMEM((H * 4, 8, 128), jnp.float32),
        pltpu.VMEM((H * 4, 8, 128), jnp.float32),
        pltpu.VMEM((16, 8, 128), jnp.float32),
        pltpu.VMEM((BPW,), jnp.float32),
        pltpu.SemaphoreType.DMA,
    ],
)
def _dist_kernel(rep_t_hbm, idx_hbm, table3_hbm, out_hbm,
                 idx_v, buf0, buf1, rep_v, out_v, sem):
    wid = lax.axis_index("s") * NC + lax.axis_index("c")
    base = wid * BPW

    pltpu.sync_copy(idx_hbm.at[pl.ds(base, BPW)], idx_v)
    for rb in range(4):
        for cb in range(4):
            pltpu.sync_copy(
                rep_t_hbm.at[pl.ds(rb * 8, 8), pl.ds(base + cb * 128, 128)],
                rep_v.at[rb * 4 + cb],
            )

    lane = lax.iota(jnp.int32, L)
    lane_lo = lane < jnp.int32(8)
    bufs = (buf0, buf1)

    def fire_half(buf, vec16, half):
        for j in range(H):
            va = pl.multiple_of(
                lax.shift_left(
                    lax.shift_right_logical(vec16[half * 8 + j], 7), 7),
                128)
            pltpu.async_copy(
                table3_hbm.at[:, :, pl.ds(va, 128)],
                buf.at[pl.ds(j * 4, 4)],
                sem,
            )

    def wait_half(buf):
        for j in range(H):
            pltpu.make_async_copy(
                table3_hbm.at[:, :, pl.ds(0, 128)],
                buf.at[pl.ds(j * 4, 4)],
                sem,
            ).wait()

    tiles_lo = lax.bitwise_and(lane, jnp.int32(7)) * 4

    vec_p0 = idx_v[pl.ds(0, L)]
    fire_half(buf0, vec_p0, 0)
    fire_half(buf1, vec_p0, 1)

    def super_step(s, carry):
        s0 = s * H * HPS
        acc_keep = jnp.zeros((L,), jnp.float32)
        rv_keep = [jnp.zeros((L,), jnp.float32)] * REPR
        for hu in range(HPS):
            p = hu // 2
            half = hu % 2
            buf = bufs[hu % 2]
            wait_half(buf)
            vec16 = idx_v[pl.ds(s0 + p * L, L)]
            off = lax.bitwise_and(vec16, jnp.int32(127))
            if half == 0:
                rv_keep = [
                    rep_v[(d // 8) * 4 + s, d % 8, pl.ds(p * L, L)]
                    for d in range(REPR)
                ]
            acc = jnp.full((L,), 1e-12, jnp.float32)
            for d in range(REPR):
                tile = tiles_lo + jnp.int32(d // 8)
                r = jnp.full((L,), d % 8, jnp.int32)
                e = plsc.load_gather(buf, [tile, r, off])
                df = e - rv_keep[d]
                acc = acc + df * df
            if half == 0:
                acc_keep = acc
            else:
                accp = jnp.where(lane_lo, acc_keep, acc)
                i = lax.bitcast_convert_type(accp, jnp.int32)
                i = 0x5F3759DF - lax.shift_right_logical(i, 1)
                y = lax.bitcast_convert_type(i, jnp.float32)
                halfv = accp * jnp.float32(0.5)
                for _ in range(3):
                    y = y * (jnp.float32(1.5) - halfv * y * y)
                out_v[pl.ds(s0 + p * L, L)] = accp * y
            nh = hu + 2
            if nh < HPS:
                nvec = idx_v[pl.ds(s0 + (nh // 2) * L, L)]
                fire_half(buf, nvec, nh % 2)
            else:
                @pl.when(s < NSUP - 1)
                def _():
                    nvec = idx_v[pl.ds(s0 + H * HPS + ((nh - HPS) // 2) * L, L)]
                    fire_half(buf, nvec, nh % 2)
        return carry

    lax.fori_loop(0, NSUP, super_step, 0)
    pltpu.sync_copy(out_v, out_hbm.at[pl.ds(base, BPW)])


def kernel(rep, oracle_structure, embedding_weight):
    idx = oracle_structure.astype(jnp.int32)
    table3 = embedding_weight.T.reshape(4, 8, 1000000)
    return _dist_kernel(rep.T, idx, table3)
